# Initial kernel scaffold; baseline (speedup 1.0000x reference)
import jax
import jax.numpy as jnp
from jax.experimental import pallas as pl


def kernel(x, eid, W, b):
    raise NotImplementedError("write your pallas kernel here")


# speedup vs baseline: 2.5573x; 2.5573x over previous
import functools

import jax
import jax.numpy as jnp
from jax.experimental import pallas as pl
from jax.experimental.pallas import tpu as pltpu

E = 8
B = 32
T = 100
P = 2048
N = 1024


def _linear_kernel(se_ref, pm_ref, x_ref, w_ref, b_ref, o_ref):
    del se_ref, pm_ref
    acc = jax.lax.dot_general(
        x_ref[0], w_ref[0],
        dimension_numbers=(((1,), (1,)), ((), ())),
        preferred_element_type=jnp.float32,
    )
    o_ref[0] = acc + b_ref[0]


def kernel(x, eid, W, b):
    x = x.reshape(B, T, P)
    perm = jnp.argsort(eid, stable=True).astype(jnp.int32)
    sorted_eid = jnp.take(eid, perm)
    b3 = b.reshape(E, 1, N)

    grid_spec = pltpu.PrefetchScalarGridSpec(
        num_scalar_prefetch=2,
        grid=(B,),
        in_specs=[
            pl.BlockSpec((1, T, P), lambda i, se, pm: (pm[i], 0, 0)),
            pl.BlockSpec((1, N, P), lambda i, se, pm: (se[i], 0, 0)),
            pl.BlockSpec((1, 1, N), lambda i, se, pm: (se[i], 0, 0)),
        ],
        out_specs=pl.BlockSpec((1, T, N), lambda i, se, pm: (pm[i], 0, 0)),
    )
    out = pl.pallas_call(
        _linear_kernel,
        grid_spec=grid_spec,
        out_shape=jax.ShapeDtypeStruct((B, T, N), jnp.float32),
    )(sorted_eid, perm, x, W, b3)
    return out


# speedup vs baseline: 2.5678x; 1.0041x over previous
import functools

import jax
import jax.numpy as jnp
from jax.experimental import pallas as pl
from jax.experimental.pallas import tpu as pltpu

E = 8
B = 32
T = 100
P = 2048
N = 1024


def _linear_kernel(se_ref, pm_ref, x_ref, w_ref, b_ref, o_ref):
    del se_ref, pm_ref
    acc = jax.lax.dot_general(
        x_ref[0], w_ref[0],
        dimension_numbers=(((1,), (1,)), ((), ())),
        preferred_element_type=jnp.float32,
    )
    o_ref[0] = acc + b_ref[0]


def kernel(x, eid, W, b):
    x = x.reshape(B, T, P)
    iota = jnp.arange(B, dtype=jnp.int32)
    lt = (eid[None, :] < eid[:, None]) | (
        (eid[None, :] == eid[:, None]) & (iota[None, :] < iota[:, None])
    )
    rank = jnp.sum(lt.astype(jnp.int32), axis=1)
    onehot = (rank[None, :] == iota[:, None]).astype(jnp.int32)
    perm = onehot @ iota
    sorted_eid = onehot @ eid
    b3 = b.reshape(E, 1, N)

    grid_spec = pltpu.PrefetchScalarGridSpec(
        num_scalar_prefetch=2,
        grid=(B,),
        in_specs=[
            pl.BlockSpec((1, T, P), lambda i, se, pm: (pm[i], 0, 0)),
            pl.BlockSpec((1, N, P), lambda i, se, pm: (se[i], 0, 0)),
            pl.BlockSpec((1, 1, N), lambda i, se, pm: (se[i], 0, 0)),
        ],
        out_specs=pl.BlockSpec((1, T, N), lambda i, se, pm: (pm[i], 0, 0)),
    )
    out = pl.pallas_call(
        _linear_kernel,
        grid_spec=grid_spec,
        out_shape=jax.ShapeDtypeStruct((B, T, N), jnp.float32),
    )(sorted_eid, perm, x, W, b3)
    return out
